# baseline (speedup 1.0000x reference)
import functools

import jax
import jax.numpy as jnp
from jax.experimental import pallas as pl

N = 10000
TM = 400


def _gcn_kern(a_ref, v_ref, w_ref, b_ref, o_ref):
    av = jnp.dot(a_ref[...], v_ref[...], preferred_element_type=jnp.float32)
    o_ref[...] = jnp.maximum(
        jnp.dot(av, w_ref[...], preferred_element_type=jnp.float32) + b_ref[...],
        0.0,
    )


def _gcn_layer(A, v, W, b):
    din, dout = W.shape
    return pl.pallas_call(
        _gcn_kern,
        grid=(N // TM,),
        in_specs=[
            pl.BlockSpec((TM, N), lambda i: (i, 0)),
            pl.BlockSpec((N, din), lambda i: (0, 0)),
            pl.BlockSpec((din, dout), lambda i: (0, 0)),
            pl.BlockSpec((1, dout), lambda i: (0, 0)),
        ],
        out_specs=pl.BlockSpec((TM, dout), lambda i: (i, 0)),
        out_shape=jax.ShapeDtypeStruct((N, dout), jnp.float32),
    )(A, v, W, b.reshape(1, dout))


def _gcn_pool_kern(a_ref, v_ref, w_ref, b_ref, o_ref):
    av = jnp.dot(a_ref[...], v_ref[...], preferred_element_type=jnp.float32)
    act = jnp.maximum(
        jnp.dot(av, w_ref[...], preferred_element_type=jnp.float32) + b_ref[...],
        0.0,
    )
    part = jnp.sum(act, axis=0, keepdims=True)

    @pl.when(pl.program_id(0) == 0)
    def _():
        o_ref[...] = jnp.zeros_like(o_ref)

    o_ref[...] += part


def _gcn_layer_pooled(A, v, W, b):
    din, dout = W.shape
    return pl.pallas_call(
        _gcn_pool_kern,
        grid=(N // TM,),
        in_specs=[
            pl.BlockSpec((TM, N), lambda i: (i, 0)),
            pl.BlockSpec((N, din), lambda i: (0, 0)),
            pl.BlockSpec((din, dout), lambda i: (0, 0)),
            pl.BlockSpec((1, dout), lambda i: (0, 0)),
        ],
        out_specs=pl.BlockSpec((1, dout), lambda i: (0, 0)),
        out_shape=jax.ShapeDtypeStruct((1, dout), jnp.float32),
    )(A, v, W, b.reshape(1, dout))


def _head_kern(x_ref, d1_ref, c1_ref, d2_ref, c2_ref, d3_ref, c3_ref, o_ref):
    x = x_ref[...]
    nrm = jnp.maximum(jnp.sqrt(jnp.sum(x * x)), 1e-12)
    x = x / nrm
    x = jnp.maximum(jnp.dot(x, d1_ref[...], preferred_element_type=jnp.float32) + c1_ref[...], 0.0)
    x = jnp.maximum(jnp.dot(x, d2_ref[...], preferred_element_type=jnp.float32) + c2_ref[...], 0.0)
    o_ref[...] = jnp.dot(x, d3_ref[...], preferred_element_type=jnp.float32) + c3_ref[...]


def _head(x, D1, c1, D2, c2, D3, c3):
    return pl.pallas_call(
        _head_kern,
        out_shape=jax.ShapeDtypeStruct((1, 1), jnp.float32),
    )(x, D1, c1.reshape(1, -1), D2, c2.reshape(1, -1), D3, c3.reshape(1, -1))


def kernel(V, A, W1, b1, W2, b2, W3, b3, W4, b4, W5, b5, D1, c1, D2, c2, D3, c3):
    v = _gcn_layer(A, V, W1, b1)
    v = _gcn_layer(A, v, W2, b2)
    v = _gcn_layer(A, v, W3, b3)
    v = _gcn_layer(A, v, W4, b4)
    pooled = _gcn_layer_pooled(A, v, W5, b5)
    y = _head(pooled, D1, c1, D2, c2, D3, c3)
    return jnp.squeeze(y, axis=1)


# speedup vs baseline: 1.2292x; 1.2292x over previous
import jax
import jax.numpy as jnp
from jax.experimental import pallas as pl

N = 10000
TM1 = 200
TM = 400


def _layer1_conv_kern(a_ref, v_ref, w_ref, b_ref, o_ref, a16_ref):
    a16 = a_ref[...].astype(jnp.bfloat16)
    a16_ref[...] = a16
    av = jnp.dot(a16, v_ref[...], preferred_element_type=jnp.float32)
    o_ref[...] = jnp.maximum(
        jnp.dot(av, w_ref[...], preferred_element_type=jnp.float32) + b_ref[...],
        0.0,
    ).astype(jnp.bfloat16)


def _layer1_conv(A, v16, W, b):
    din, dout = W.shape
    return pl.pallas_call(
        _layer1_conv_kern,
        grid=(N // TM1,),
        in_specs=[
            pl.BlockSpec((TM1, N), lambda i: (i, 0)),
            pl.BlockSpec((N, din), lambda i: (0, 0)),
            pl.BlockSpec((din, dout), lambda i: (0, 0)),
            pl.BlockSpec((1, dout), lambda i: (0, 0)),
        ],
        out_specs=[
            pl.BlockSpec((TM1, dout), lambda i: (i, 0)),
            pl.BlockSpec((TM1, N), lambda i: (i, 0)),
        ],
        out_shape=[
            jax.ShapeDtypeStruct((N, dout), jnp.bfloat16),
            jax.ShapeDtypeStruct((N, N), jnp.bfloat16),
        ],
    )(A, v16, W, b.reshape(1, dout))


def _gcn_kern(a_ref, v_ref, w_ref, b_ref, o_ref):
    av = jnp.dot(a_ref[...], v_ref[...], preferred_element_type=jnp.float32)
    o_ref[...] = jnp.maximum(
        jnp.dot(av, w_ref[...], preferred_element_type=jnp.float32) + b_ref[...],
        0.0,
    ).astype(jnp.bfloat16)


def _gcn_layer(A16, v, W, b):
    din, dout = W.shape
    return pl.pallas_call(
        _gcn_kern,
        grid=(N // TM,),
        in_specs=[
            pl.BlockSpec((TM, N), lambda i: (i, 0)),
            pl.BlockSpec((N, din), lambda i: (0, 0)),
            pl.BlockSpec((din, dout), lambda i: (0, 0)),
            pl.BlockSpec((1, dout), lambda i: (0, 0)),
        ],
        out_specs=pl.BlockSpec((TM, dout), lambda i: (i, 0)),
        out_shape=jax.ShapeDtypeStruct((N, dout), jnp.bfloat16),
    )(A16, v, W, b.reshape(1, dout))


def _gcn_pool_kern(a_ref, v_ref, w_ref, b_ref, o_ref):
    av = jnp.dot(a_ref[...], v_ref[...], preferred_element_type=jnp.float32)
    act = jnp.maximum(
        jnp.dot(av, w_ref[...], preferred_element_type=jnp.float32) + b_ref[...],
        0.0,
    )
    part = jnp.sum(act, axis=0, keepdims=True)

    @pl.when(pl.program_id(0) == 0)
    def _():
        o_ref[...] = jnp.zeros_like(o_ref)

    o_ref[...] += part


def _gcn_layer_pooled(A16, v, W, b):
    din, dout = W.shape
    return pl.pallas_call(
        _gcn_pool_kern,
        grid=(N // TM,),
        in_specs=[
            pl.BlockSpec((TM, N), lambda i: (i, 0)),
            pl.BlockSpec((N, din), lambda i: (0, 0)),
            pl.BlockSpec((din, dout), lambda i: (0, 0)),
            pl.BlockSpec((1, dout), lambda i: (0, 0)),
        ],
        out_specs=pl.BlockSpec((1, dout), lambda i: (0, 0)),
        out_shape=jax.ShapeDtypeStruct((1, dout), jnp.float32),
    )(A16, v, W, b.reshape(1, dout))


def _head_kern(x_ref, d1_ref, c1_ref, d2_ref, c2_ref, d3_ref, c3_ref, o_ref):
    x = x_ref[...]
    nrm = jnp.maximum(jnp.sqrt(jnp.sum(x * x)), 1e-12)
    x = x / nrm
    x = jnp.maximum(jnp.dot(x, d1_ref[...], preferred_element_type=jnp.float32) + c1_ref[...], 0.0)
    x = jnp.maximum(jnp.dot(x, d2_ref[...], preferred_element_type=jnp.float32) + c2_ref[...], 0.0)
    o_ref[...] = jnp.dot(x, d3_ref[...], preferred_element_type=jnp.float32) + c3_ref[...]


def _head(x, D1, c1, D2, c2, D3, c3):
    return pl.pallas_call(
        _head_kern,
        out_shape=jax.ShapeDtypeStruct((1, 1), jnp.float32),
    )(x, D1, c1.reshape(1, -1), D2, c2.reshape(1, -1), D3, c3.reshape(1, -1))


def kernel(V, A, W1, b1, W2, b2, W3, b3, W4, b4, W5, b5, D1, c1, D2, c2, D3, c3):
    v, A16 = _layer1_conv(A, V.astype(jnp.bfloat16), W1.astype(jnp.bfloat16), b1)
    v = _gcn_layer(A16, v, W2.astype(jnp.bfloat16), b2)
    v = _gcn_layer(A16, v, W3.astype(jnp.bfloat16), b3)
    v = _gcn_layer(A16, v, W4.astype(jnp.bfloat16), b4)
    pooled = _gcn_layer_pooled(A16, v, W5.astype(jnp.bfloat16), b5)
    y = _head(pooled, D1, c1, D2, c2, D3, c3)
    return jnp.squeeze(y, axis=1)


# speedup vs baseline: 1.4752x; 1.2001x over previous
import jax
import jax.numpy as jnp
from jax.experimental import pallas as pl

F8 = jnp.float8_e4m3fn
N = 10000
TM1 = 200
TM = 400


def _layer1_conv_kern(a_ref, v_ref, w_ref, b_ref, o_ref, a8_ref):
    a = a_ref[...]
    a8_ref[...] = a.astype(F8)
    av = jnp.dot(a.astype(jnp.bfloat16), v_ref[...], preferred_element_type=jnp.float32)
    o_ref[...] = jnp.maximum(
        jnp.dot(av, w_ref[...], preferred_element_type=jnp.float32) + b_ref[...],
        0.0,
    ).astype(jnp.bfloat16)


def _layer1_conv(A, v16, W, b):
    din, dout = W.shape
    return pl.pallas_call(
        _layer1_conv_kern,
        grid=(N // TM1,),
        in_specs=[
            pl.BlockSpec((TM1, N), lambda i: (i, 0)),
            pl.BlockSpec((N, din), lambda i: (0, 0)),
            pl.BlockSpec((din, dout), lambda i: (0, 0)),
            pl.BlockSpec((1, dout), lambda i: (0, 0)),
        ],
        out_specs=[
            pl.BlockSpec((TM1, dout), lambda i: (i, 0)),
            pl.BlockSpec((TM1, N), lambda i: (i, 0)),
        ],
        out_shape=[
            jax.ShapeDtypeStruct((N, dout), jnp.bfloat16),
            jax.ShapeDtypeStruct((N, N), F8),
        ],
    )(A, v16, W, b.reshape(1, dout))


def _gcn_kern(a_ref, v_ref, w_ref, b_ref, o_ref):
    av = jnp.dot(a_ref[...], v_ref[...], preferred_element_type=jnp.float32)
    o_ref[...] = jnp.maximum(
        jnp.dot(av, w_ref[...], preferred_element_type=jnp.float32) + b_ref[...],
        0.0,
    ).astype(jnp.bfloat16)


def _gcn_layer(A16, v, W, b):
    din, dout = W.shape
    return pl.pallas_call(
        _gcn_kern,
        grid=(N // TM,),
        in_specs=[
            pl.BlockSpec((TM, N), lambda i: (i, 0)),
            pl.BlockSpec((N, din), lambda i: (0, 0)),
            pl.BlockSpec((din, dout), lambda i: (0, 0)),
            pl.BlockSpec((1, dout), lambda i: (0, 0)),
        ],
        out_specs=pl.BlockSpec((TM, dout), lambda i: (i, 0)),
        out_shape=jax.ShapeDtypeStruct((N, dout), jnp.bfloat16),
    )(A16, v, W, b.reshape(1, dout))


def _gcn_pool_kern(a_ref, v_ref, w_ref, b_ref, o_ref):
    av = jnp.dot(a_ref[...], v_ref[...], preferred_element_type=jnp.float32)
    act = jnp.maximum(
        jnp.dot(av, w_ref[...], preferred_element_type=jnp.float32) + b_ref[...],
        0.0,
    )
    part = jnp.sum(act, axis=0, keepdims=True)

    @pl.when(pl.program_id(0) == 0)
    def _():
        o_ref[...] = jnp.zeros_like(o_ref)

    o_ref[...] += part


def _gcn_layer_pooled(A16, v, W, b):
    din, dout = W.shape
    return pl.pallas_call(
        _gcn_pool_kern,
        grid=(N // TM,),
        in_specs=[
            pl.BlockSpec((TM, N), lambda i: (i, 0)),
            pl.BlockSpec((N, din), lambda i: (0, 0)),
            pl.BlockSpec((din, dout), lambda i: (0, 0)),
            pl.BlockSpec((1, dout), lambda i: (0, 0)),
        ],
        out_specs=pl.BlockSpec((1, dout), lambda i: (0, 0)),
        out_shape=jax.ShapeDtypeStruct((1, dout), jnp.float32),
    )(A16, v, W, b.reshape(1, dout))


def _head_kern(x_ref, d1_ref, c1_ref, d2_ref, c2_ref, d3_ref, c3_ref, o_ref):
    x = x_ref[...]
    nrm = jnp.maximum(jnp.sqrt(jnp.sum(x * x)), 1e-12)
    x = x / nrm
    x = jnp.maximum(jnp.dot(x, d1_ref[...], preferred_element_type=jnp.float32) + c1_ref[...], 0.0)
    x = jnp.maximum(jnp.dot(x, d2_ref[...], preferred_element_type=jnp.float32) + c2_ref[...], 0.0)
    o_ref[...] = jnp.dot(x, d3_ref[...], preferred_element_type=jnp.float32) + c3_ref[...]


def _head(x, D1, c1, D2, c2, D3, c3):
    return pl.pallas_call(
        _head_kern,
        out_shape=jax.ShapeDtypeStruct((1, 1), jnp.float32),
    )(x, D1, c1.reshape(1, -1), D2, c2.reshape(1, -1), D3, c3.reshape(1, -1))


def kernel(V, A, W1, b1, W2, b2, W3, b3, W4, b4, W5, b5, D1, c1, D2, c2, D3, c3):
    v, A16 = _layer1_conv(A, V.astype(jnp.bfloat16), W1.astype(jnp.bfloat16), b1)
    v = _gcn_layer(A16, v, W2.astype(jnp.bfloat16), b2)
    v = _gcn_layer(A16, v, W3.astype(jnp.bfloat16), b3)
    v = _gcn_layer(A16, v, W4.astype(jnp.bfloat16), b4)
    pooled = _gcn_layer_pooled(A16, v, W5.astype(jnp.bfloat16), b5)
    y = _head(pooled, D1, c1, D2, c2, D3, c3)
    return jnp.squeeze(y, axis=1)


# speedup vs baseline: 1.5406x; 1.0443x over previous
import jax
import jax.numpy as jnp
from jax.experimental import pallas as pl
from jax.experimental.pallas import tpu as pltpu

F8 = jnp.float8_e4m3fn
N = 10000
TM1 = 200
TM = 1000
STEPS = N // TM
D = 128


def _layer1_conv_kern(a_ref, v_ref, w_ref, b_ref, o_ref, a8_ref):
    a = a_ref[...]
    a8_ref[...] = a.astype(F8)
    av = jnp.dot(a.astype(jnp.bfloat16), v_ref[...], preferred_element_type=jnp.float32)
    o_ref[...] = jnp.maximum(
        jnp.dot(av, w_ref[...], preferred_element_type=jnp.float32) + b_ref[...],
        0.0,
    ).astype(jnp.bfloat16)


def _layer1_conv(A, v16, W, b):
    din, dout = W.shape
    return pl.pallas_call(
        _layer1_conv_kern,
        grid=(N // TM1,),
        in_specs=[
            pl.BlockSpec((TM1, N), lambda i: (i, 0)),
            pl.BlockSpec((N, din), lambda i: (0, 0)),
            pl.BlockSpec((din, dout), lambda i: (0, 0)),
            pl.BlockSpec((1, dout), lambda i: (0, 0)),
        ],
        out_specs=[
            pl.BlockSpec((TM1, dout), lambda i: (i, 0)),
            pl.BlockSpec((TM1, N), lambda i: (i, 0)),
        ],
        out_shape=[
            jax.ShapeDtypeStruct((N, dout), jnp.bfloat16),
            jax.ShapeDtypeStruct((N, N), F8),
        ],
    )(A, v16, W, b.reshape(1, dout))


def _mega_kern(a8_ref, v1_ref, ws_ref, bs_ref, d1_ref, c1_ref, d2_ref, c2_ref,
               d3_ref, c3_ref, o_ref, vs_ref, pool_ref):
    l = pl.program_id(0)
    i = pl.program_id(1)

    @pl.when((l == 0) & (i == 0))
    def _():
        vs_ref[0] = v1_ref[...]
        pool_ref[...] = jnp.zeros_like(pool_ref)

    cur = jax.lax.rem(l, 2)
    av = jnp.dot(a8_ref[...], vs_ref[cur], preferred_element_type=jnp.float32)
    act = jnp.maximum(
        jnp.dot(av, ws_ref[0], preferred_element_type=jnp.float32) + bs_ref[0],
        0.0,
    )

    @pl.when(l < 3)
    def _():
        vs_ref[1 - cur, pl.ds(i * TM, TM), :] = act.astype(jnp.bfloat16)

    @pl.when(l == 3)
    def _():
        pool_ref[...] += jnp.sum(act, axis=0, keepdims=True)

    @pl.when((l == 3) & (i == STEPS - 1))
    def _():
        x = pool_ref[...]
        nrm = jnp.maximum(jnp.sqrt(jnp.sum(x * x)), 1e-12)
        x = x / nrm
        x = jnp.maximum(jnp.dot(x, d1_ref[...], preferred_element_type=jnp.float32) + c1_ref[...], 0.0)
        x = jnp.maximum(jnp.dot(x, d2_ref[...], preferred_element_type=jnp.float32) + c2_ref[...], 0.0)
        o_ref[...] = jnp.dot(x, d3_ref[...], preferred_element_type=jnp.float32) + c3_ref[...]


def _mega(A8, v1p, Ws, bs, D1, c1, D2, c2, D3, c3):
    return pl.pallas_call(
        _mega_kern,
        grid=(4, STEPS),
        in_specs=[
            pl.BlockSpec((TM, N), lambda l, i: (i, 0)),
            pl.BlockSpec((N, D), lambda l, i: (0, 0)),
            pl.BlockSpec((1, D, D), lambda l, i: (l, 0, 0)),
            pl.BlockSpec((1, 1, D), lambda l, i: (l, 0, 0)),
            pl.BlockSpec((128, 256), lambda l, i: (0, 0)),
            pl.BlockSpec((1, 256), lambda l, i: (0, 0)),
            pl.BlockSpec((256, 128), lambda l, i: (0, 0)),
            pl.BlockSpec((1, 128), lambda l, i: (0, 0)),
            pl.BlockSpec((128, 1), lambda l, i: (0, 0)),
            pl.BlockSpec((1, 1), lambda l, i: (0, 0)),
        ],
        out_specs=pl.BlockSpec((1, 1), lambda l, i: (0, 0)),
        out_shape=jax.ShapeDtypeStruct((1, 1), jnp.float32),
        scratch_shapes=[
            pltpu.VMEM((2, N, D), jnp.bfloat16),
            pltpu.VMEM((1, D), jnp.float32),
        ],
    )(A8, v1p, Ws, bs, D1, c1, D2, c2, D3, c3)


def _pad2(M, r, c):
    return jnp.pad(M, ((0, r - M.shape[0]), (0, c - M.shape[1])))


def kernel(V, A, W1, b1, W2, b2, W3, b3, W4, b4, W5, b5, D1, c1, D2, c2, D3, c3):
    W1p = _pad2(W1, 11, D)
    b1p = jnp.pad(b1, (0, D - b1.shape[0]))
    v1p, A8 = _layer1_conv(A, V.astype(jnp.bfloat16), W1p.astype(jnp.bfloat16), b1p)
    Ws = jnp.stack([_pad2(W, D, D) for W in (W2, W3, W4, W5)])
    bs = jnp.stack([jnp.pad(b, (0, D - b.shape[0])).reshape(1, D) for b in (b2, b3, b4, b5)])
    y = _mega(A8, v1p, Ws, bs, D1, c1.reshape(1, -1), D2, c2.reshape(1, -1),
              D3, c3.reshape(1, -1))
    return jnp.squeeze(y, axis=1)


# speedup vs baseline: 1.7673x; 1.1472x over previous
import jax
import jax.numpy as jnp
from jax.experimental import pallas as pl
from jax.experimental.pallas import tpu as pltpu

F8 = jnp.float8_e4m3fn
FMAX = 240.0
N = 10000
TM1 = 200
TM = 400
STEPS = N // TM
D = 128


def _layer1_conv_kern(a_ref, v_ref, w_ref, b_ref, o_ref, a8_ref):
    a16 = a_ref[...].astype(jnp.bfloat16)
    a8_ref[...] = a16.astype(F8)
    av = jnp.dot(a16, v_ref[...], preferred_element_type=jnp.float32)
    o_ref[...] = jnp.maximum(
        jnp.dot(av, w_ref[...], preferred_element_type=jnp.float32) + b_ref[...],
        0.0,
    ).astype(jnp.bfloat16)


def _layer1_conv(A, v16, W, b):
    din, dout = W.shape
    return pl.pallas_call(
        _layer1_conv_kern,
        grid=(N // TM1,),
        in_specs=[
            pl.BlockSpec((TM1, N), lambda i: (i, 0)),
            pl.BlockSpec((N, din), lambda i: (0, 0)),
            pl.BlockSpec((din, dout), lambda i: (0, 0)),
            pl.BlockSpec((1, dout), lambda i: (0, 0)),
        ],
        out_specs=[
            pl.BlockSpec((TM1, dout), lambda i: (i, 0)),
            pl.BlockSpec((TM1, N), lambda i: (i, 0)),
        ],
        out_shape=[
            jax.ShapeDtypeStruct((N, dout), jnp.bfloat16),
            jax.ShapeDtypeStruct((N, N), F8),
        ],
    )(A, v16, W, b.reshape(1, dout))


def _mega_kern(a8_ref, v1_ref, ws_ref, bs_ref, d1_ref, c1_ref, d2_ref, c2_ref,
               d3_ref, c3_ref, o_ref, v8_ref, stage_ref, m_ref, sc_ref, pool_ref):
    l = pl.program_id(0)
    i = pl.program_id(1)

    @pl.when((l == 0) & (i == 0))
    def _():
        m = jnp.maximum(jnp.max(v1_ref[...].astype(jnp.float32)), 1e-30)
        v8_ref[...] = (v1_ref[...].astype(jnp.float32) * (FMAX / m)).astype(F8)
        sc_ref[0, 0] = m / FMAX
        m_ref[...] = jnp.zeros_like(m_ref)
        pool_ref[...] = jnp.zeros_like(pool_ref)

    @pl.when((l > 0) & (i == 0))
    def _():
        m = jnp.maximum(jnp.max(m_ref[...]), 1e-30)
        v8_ref[...] = (stage_ref[...].astype(jnp.float32) * (FMAX / m)).astype(F8)
        sc_ref[0, 0] = m / FMAX
        m_ref[...] = jnp.zeros_like(m_ref)

    av = jnp.dot(a8_ref[...], v8_ref[...], preferred_element_type=jnp.float32)
    w = ws_ref[0] * sc_ref[0, 0]
    act = jnp.maximum(
        jnp.dot(av, w, preferred_element_type=jnp.float32) + bs_ref[0],
        0.0,
    )

    @pl.when(l < 3)
    def _():
        stage_ref[pl.ds(i * TM, TM), :] = act.astype(jnp.bfloat16)
        m_ref[...] = jnp.maximum(m_ref[...], jnp.max(act, axis=0, keepdims=True))

    @pl.when(l == 3)
    def _():
        pool_ref[...] += jnp.sum(act, axis=0, keepdims=True)

    @pl.when((l == 3) & (i == STEPS - 1))
    def _():
        x = pool_ref[...]
        nrm = jnp.maximum(jnp.sqrt(jnp.sum(x * x)), 1e-12)
        x = x / nrm
        x = jnp.maximum(jnp.dot(x, d1_ref[...], preferred_element_type=jnp.float32) + c1_ref[...], 0.0)
        x = jnp.maximum(jnp.dot(x, d2_ref[...], preferred_element_type=jnp.float32) + c2_ref[...], 0.0)
        o_ref[...] = jnp.dot(x, d3_ref[...], preferred_element_type=jnp.float32) + c3_ref[...]


def _mega(A8, v1p, Ws, bs, D1, c1, D2, c2, D3, c3):
    return pl.pallas_call(
        _mega_kern,
        grid=(4, STEPS),
        in_specs=[
            pl.BlockSpec((TM, N), lambda l, i: (i, 0)),
            pl.BlockSpec((N, D), lambda l, i: (0, 0)),
            pl.BlockSpec((1, D, D), lambda l, i: (l, 0, 0)),
            pl.BlockSpec((1, 1, D), lambda l, i: (l, 0, 0)),
            pl.BlockSpec((128, 256), lambda l, i: (0, 0)),
            pl.BlockSpec((1, 256), lambda l, i: (0, 0)),
            pl.BlockSpec((256, 128), lambda l, i: (0, 0)),
            pl.BlockSpec((1, 128), lambda l, i: (0, 0)),
            pl.BlockSpec((128, 1), lambda l, i: (0, 0)),
            pl.BlockSpec((1, 1), lambda l, i: (0, 0)),
        ],
        out_specs=pl.BlockSpec((1, 1), lambda l, i: (0, 0)),
        out_shape=jax.ShapeDtypeStruct((1, 1), jnp.float32),
        scratch_shapes=[
            pltpu.VMEM((N, D), F8),
            pltpu.VMEM((N, D), jnp.bfloat16),
            pltpu.VMEM((1, D), jnp.float32),
            pltpu.SMEM((1, 1), jnp.float32),
            pltpu.VMEM((1, D), jnp.float32),
        ],
    )(A8, v1p, Ws, bs, D1, c1, D2, c2, D3, c3)


def _pad2(M, r, c):
    return jnp.pad(M, ((0, r - M.shape[0]), (0, c - M.shape[1])))


def kernel(V, A, W1, b1, W2, b2, W3, b3, W4, b4, W5, b5, D1, c1, D2, c2, D3, c3):
    W1p = _pad2(W1, 11, D)
    b1p = jnp.pad(b1, (0, D - b1.shape[0]))
    v1p, A8 = _layer1_conv(A, V.astype(jnp.bfloat16), W1p.astype(jnp.bfloat16), b1p)
    Ws = jnp.stack([_pad2(W, D, D) for W in (W2, W3, W4, W5)])
    bs = jnp.stack([jnp.pad(b, (0, D - b.shape[0])).reshape(1, D) for b in (b2, b3, b4, b5)])
    y = _mega(A8, v1p, Ws, bs, D1, c1.reshape(1, -1), D2, c2.reshape(1, -1),
              D3, c3.reshape(1, -1))
    return jnp.squeeze(y, axis=1)


# speedup vs baseline: 1.9623x; 1.1103x over previous
import jax
import jax.numpy as jnp
from jax.experimental import pallas as pl
from jax.experimental.pallas import tpu as pltpu

F8 = jnp.float8_e4m3fn
FMAX = 240.0
N = 10000
TM1 = 400
TM = 1000
STEPS = N // TM
D = 128


def _layer1_conv_kern(a_ref, v_ref, w_ref, b_ref, o_ref, a8_ref):
    a16 = a_ref[...].astype(jnp.bfloat16)
    a8_ref[...] = a16.astype(F8)
    av = jnp.dot(a16, v_ref[...], preferred_element_type=jnp.float32)
    o_ref[...] = jnp.maximum(
        jnp.dot(av, w_ref[...], preferred_element_type=jnp.float32) + b_ref[...],
        0.0,
    ).astype(jnp.bfloat16)


def _layer1_conv(A, v16, W, b):
    din, dout = W.shape
    return pl.pallas_call(
        _layer1_conv_kern,
        grid=(N // TM1,),
        in_specs=[
            pl.BlockSpec((TM1, N), lambda i: (i, 0)),
            pl.BlockSpec((N, din), lambda i: (0, 0)),
            pl.BlockSpec((din, dout), lambda i: (0, 0)),
            pl.BlockSpec((1, dout), lambda i: (0, 0)),
        ],
        out_specs=[
            pl.BlockSpec((TM1, dout), lambda i: (i, 0)),
            pl.BlockSpec((TM1, N), lambda i: (i, 0)),
        ],
        out_shape=[
            jax.ShapeDtypeStruct((N, dout), jnp.bfloat16),
            jax.ShapeDtypeStruct((N, N), F8),
        ],
    )(A, v16, W, b.reshape(1, dout))


def _mega_kern(a8_ref, v1_ref, ws_ref, bs_ref, d1_ref, c1_ref, d2_ref, c2_ref,
               d3_ref, c3_ref, o_ref, v8_ref, stage_ref, m_ref, sc_ref, pool_ref):
    l = pl.program_id(0)
    i = pl.program_id(1)

    @pl.when((l == 0) & (i == 0))
    def _():
        m = jnp.maximum(jnp.max(v1_ref[...].astype(jnp.float32)), 1e-30)
        v8_ref[...] = (v1_ref[...].astype(jnp.float32) * (FMAX / m)).astype(F8)
        sc_ref[0, 0] = m / FMAX
        m_ref[...] = jnp.zeros_like(m_ref)
        pool_ref[...] = jnp.zeros_like(pool_ref)

    @pl.when((l > 0) & (i == 0))
    def _():
        m = jnp.maximum(jnp.max(m_ref[...]), 1e-30)
        v8_ref[...] = (stage_ref[...] * (FMAX / m)).astype(F8)
        sc_ref[0, 0] = m / FMAX
        m_ref[...] = jnp.zeros_like(m_ref)

    av = jnp.dot(a8_ref[...], v8_ref[...], preferred_element_type=jnp.float32)
    w = ws_ref[0] * sc_ref[0, 0]
    act = jnp.maximum(
        jnp.dot(av, w, preferred_element_type=jnp.float32) + bs_ref[0],
        0.0,
    )

    @pl.when(l < 3)
    def _():
        stage_ref[pl.ds(i * TM, TM), :] = act
        m_ref[...] = jnp.maximum(m_ref[...], jnp.max(act, axis=0, keepdims=True))

    @pl.when(l == 3)
    def _():
        pool_ref[...] += jnp.sum(act, axis=0, keepdims=True)

    @pl.when((l == 3) & (i == STEPS - 1))
    def _():
        x = pool_ref[...]
        nrm = jnp.maximum(jnp.sqrt(jnp.sum(x * x)), 1e-12)
        x = x / nrm
        x = jnp.maximum(jnp.dot(x, d1_ref[...], preferred_element_type=jnp.float32) + c1_ref[...], 0.0)
        x = jnp.maximum(jnp.dot(x, d2_ref[...], preferred_element_type=jnp.float32) + c2_ref[...], 0.0)
        o_ref[...] = jnp.dot(x, d3_ref[...], preferred_element_type=jnp.float32) + c3_ref[...]


def _mega(A8, v1p, Ws, bs, D1, c1, D2, c2, D3, c3):
    return pl.pallas_call(
        _mega_kern,
        grid=(4, STEPS),
        in_specs=[
            pl.BlockSpec((TM, N), lambda l, i: (i, 0)),
            pl.BlockSpec((N, D), lambda l, i: (0, 0)),
            pl.BlockSpec((1, D, D), lambda l, i: (l, 0, 0)),
            pl.BlockSpec((1, 1, D), lambda l, i: (l, 0, 0)),
            pl.BlockSpec((128, 256), lambda l, i: (0, 0)),
            pl.BlockSpec((1, 256), lambda l, i: (0, 0)),
            pl.BlockSpec((256, 128), lambda l, i: (0, 0)),
            pl.BlockSpec((1, 128), lambda l, i: (0, 0)),
            pl.BlockSpec((128, 1), lambda l, i: (0, 0)),
            pl.BlockSpec((1, 1), lambda l, i: (0, 0)),
        ],
        out_specs=pl.BlockSpec((1, 1), lambda l, i: (0, 0)),
        out_shape=jax.ShapeDtypeStruct((1, 1), jnp.float32),
        scratch_shapes=[
            pltpu.VMEM((N, D), F8),
            pltpu.VMEM((N, D), jnp.float32),
            pltpu.VMEM((1, D), jnp.float32),
            pltpu.SMEM((1, 1), jnp.float32),
            pltpu.VMEM((1, D), jnp.float32),
        ],
    )(A8, v1p, Ws, bs, D1, c1, D2, c2, D3, c3)


def _pad2(M, r, c):
    return jnp.pad(M, ((0, r - M.shape[0]), (0, c - M.shape[1])))


def kernel(V, A, W1, b1, W2, b2, W3, b3, W4, b4, W5, b5, D1, c1, D2, c2, D3, c3):
    W1p = _pad2(W1, 11, D)
    b1p = jnp.pad(b1, (0, D - b1.shape[0]))
    v1p, A8 = _layer1_conv(A, V.astype(jnp.bfloat16), W1p.astype(jnp.bfloat16), b1p)
    Ws = jnp.stack([_pad2(W, D, D) for W in (W2, W3, W4, W5)])
    bs = jnp.stack([jnp.pad(b, (0, D - b.shape[0])).reshape(1, D) for b in (b2, b3, b4, b5)])
    y = _mega(A8, v1p, Ws, bs, D1, c1.reshape(1, -1), D2, c2.reshape(1, -1),
              D3, c3.reshape(1, -1))
    return jnp.squeeze(y, axis=1)
